# baseline (speedup 1.0000x reference)
import functools

import jax
import jax.numpy as jnp
from jax import lax
from jax.experimental import pallas as pl
from jax.experimental.pallas import tpu as pltpu
from jax.experimental.pallas import tpu_sc as plsc

_LANES = 16
_UNROLL = 2


def _gather_fn(n_total, d_model, table_words, n_cores, n_subcores, chunk,
               n_chunks):
    n_workers = n_cores * n_subcores
    n_per_w = n_total // n_workers
    chunk_words = chunk * d_model
    vecs_per_row = d_model // _LANES

    mesh = plsc.VectorSubcoreMesh(core_axis_name="c", subcore_axis_name="s")

    @functools.partial(
        pl.kernel,
        out_type=jax.ShapeDtypeStruct((n_total * d_model,), jnp.float32),
        mesh=mesh,
        scratch_types=[
            pltpu.VMEM((table_words,), jnp.float32),
            pltpu.VMEM((2 * chunk_words,), jnp.float32),
            pltpu.VMEM((2 * chunk,), jnp.int32),
            pltpu.SemaphoreType.DMA((2,)),
            pltpu.SemaphoreType.DMA((2,)),
        ],
    )
    def run(idx_hbm, table_hbm, out_hbm, table_v, rows_v, idx_s, sem_i,
            sem_s):
        wid = lax.axis_index("s") * n_cores + lax.axis_index("c")
        base = wid * n_per_w

        pltpu.sync_copy(table_hbm, table_v)

        def idx_copy(t, islot):
            return pltpu.make_async_copy(
                idx_hbm.at[pl.ds(base + t * chunk, chunk)],
                idx_s.at[pl.ds(islot * chunk, chunk)],
                sem_i.at[islot],
            )

        def store(t, slot):
            return pltpu.make_async_copy(
                rows_v.at[pl.ds(slot * chunk_words, chunk_words)],
                out_hbm.at[pl.ds((base + t * chunk) * d_model, chunk_words)],
                sem_s.at[slot],
            )

        idx_copy(0, 0).start()

        def chunk_body(t, carry):
            slot = lax.rem(t, 2)
            nslot = lax.rem(t + 1, 2)

            @pl.when(t + 1 < n_chunks)
            def _():
                idx_copy(t + 1, nslot).start()

            idx_copy(t, slot).wait()

            @pl.when(t >= 2)
            def _():
                store(t - 2, slot).wait()

            ibase = slot * chunk
            obase = slot * chunk_words

            def group_body(g2, carry2):
                ivec = idx_s[pl.ds(ibase + g2 * _LANES, _LANES)]
                for u in range(_LANES):
                    src = ivec[u] * d_model
                    dst = obase + (g2 * _LANES + u) * d_model
                    for k in range(vecs_per_row):
                        rows_v[pl.ds(dst + k * _LANES, _LANES)] = (
                            table_v[pl.ds(src + k * _LANES, _LANES)])
                return carry2

            lax.fori_loop(0, chunk // _LANES, group_body, 0)

            store(t, slot).start()
            return carry

        lax.fori_loop(0, n_chunks, chunk_body, 0)

        store(n_chunks - 2, lax.rem(n_chunks - 2, 2)).wait()
        store(n_chunks - 1, lax.rem(n_chunks - 1, 2)).wait()

    return run


def kernel(positions, pe):
    b, s = positions.shape
    v, d = pe.shape
    n_total = b * s
    idx_flat = positions.reshape(n_total).astype(jnp.int32)
    table_flat = pe.reshape(v * d)

    info = plsc.get_sparse_core_info()
    n_cores, n_subcores = info.num_cores, info.num_subcores
    n_workers = n_cores * n_subcores
    n_per_w = n_total // n_workers
    chunk = 128
    n_chunks = n_per_w // chunk

    out = _gather_fn(n_total, d, v * d, n_cores, n_subcores, chunk,
                     n_chunks)(idx_flat, table_flat)
    return out.reshape(b, s, d)


# speedup vs baseline: 4.0867x; 4.0867x over previous
import functools

import jax
import jax.numpy as jnp
from jax import lax
from jax.experimental import pallas as pl
from jax.experimental.pallas import tpu as pltpu
from jax.experimental.pallas import tpu_sc as plsc

_NSLOT = 5
_DEPTH = 3


def _gather_fn(n_total, n_vocab, d_model, n_cores, n_subcores, chunk,
               n_chunks):
    n_workers = n_cores * n_subcores
    n_per_w = n_total // n_workers

    mesh = plsc.VectorSubcoreMesh(core_axis_name="c", subcore_axis_name="s")

    @functools.partial(
        pl.kernel,
        out_type=jax.ShapeDtypeStruct((n_total, d_model), jnp.float32),
        mesh=mesh,
        scratch_types=[
            pltpu.VMEM_SHARED((n_vocab, d_model), jnp.float32),
            pltpu.VMEM((n_per_w,), jnp.int32),
            pltpu.VMEM((_NSLOT, chunk, d_model), jnp.float32),
            pltpu.SemaphoreType.DMA((_NSLOT,)),
            pltpu.SemaphoreType.DMA((_NSLOT,)),
        ],
    )
    def run(idx_hbm, table_hbm, out_hbm, table_s, idx_v, rows_v, sem_g,
            sem_s):
        sid = lax.axis_index("s")
        wid = sid * n_cores + lax.axis_index("c")
        base = wid * n_per_w

        @pl.when(sid == 0)
        def _():
            pltpu.sync_copy(table_hbm, table_s)

        pltpu.sync_copy(idx_hbm.at[pl.ds(base, n_per_w)], idx_v)
        plsc.subcore_barrier()

        def gather(i, slot):
            return pltpu.make_async_copy(
                table_s.at[idx_v.at[pl.ds(i * chunk, chunk)]],
                rows_v.at[slot],
                sem_g.at[slot],
            )

        def store(i, slot):
            return pltpu.make_async_copy(
                rows_v.at[slot],
                out_hbm.at[pl.ds(base + i * chunk, chunk)],
                sem_s.at[slot],
            )

        for b in range(_DEPTH):
            gather(b, b).start()

        for b in range(_NSLOT):
            gather(b, b).wait()
            store(b, b).start()
            nslot = (b + _DEPTH) % _NSLOT
            if b >= 2:
                store(b - 2, nslot).wait()
            gather(b + _DEPTH, nslot).start()

        def body(g, carry):
            for b in range(_NSLOT):
                i = g * _NSLOT + b
                nslot = (b + _DEPTH) % _NSLOT
                gather(i, b).wait()
                store(i, b).start()
                store(i - 2, nslot).wait()
                gather(i + _DEPTH, nslot).start()
            return carry

        lax.fori_loop(1, n_chunks // _NSLOT - 1, body, 0)

        g_last = n_chunks // _NSLOT - 1
        for b in range(_NSLOT):
            i = g_last * _NSLOT + b
            nslot = (b + _DEPTH) % _NSLOT
            gather(i, b).wait()
            store(i, b).start()
            if i + _DEPTH < n_chunks:
                store(i - 2, nslot).wait()
                gather(i + _DEPTH, nslot).start()

        for b in range(_NSLOT):
            store(g_last * _NSLOT + b, b).wait()

    return run


def kernel(positions, pe):
    b, s = positions.shape
    v, d = pe.shape
    n_total = b * s
    idx_flat = positions.reshape(n_total).astype(jnp.int32)

    info = plsc.get_sparse_core_info()
    n_cores, n_subcores = info.num_cores, info.num_subcores
    n_workers = n_cores * n_subcores
    n_per_w = n_total // n_workers
    chunk = 128
    n_chunks = n_per_w // chunk

    out = _gather_fn(n_total, v, d, n_cores, n_subcores, chunk, n_chunks)(
        idx_flat, pe
    )
    return out.reshape(b, s, d)
